# Initial kernel scaffold; baseline (speedup 1.0000x reference)
import jax
import jax.numpy as jnp
from jax.experimental import pallas as pl


def kernel(x_addr, x_tx, edge_index_input, edge_index_output, Wl1i, Wr1i, bl1i, Wl1o, Wr1o, bl1o, Wl2i, Wr2i, bl2i, Wl2o, Wr2o, bl2o, Wl3i, Wr3i, bl3i, Wl3o, Wr3o, bl3o, Wfin, bfin):
    raise NotImplementedError("write your pallas kernel here")


# speedup vs baseline: 6.9383x; 6.9383x over previous
import functools

import jax
import jax.numpy as jnp
from jax import lax
from jax.experimental import pallas as pl
from jax.experimental.pallas import tpu as pltpu
from jax.experimental.pallas import tpu_sc as plsc

N = 50000
E = 800000
D = 128
H = 32
HH = 16
OUT = 2
OUTP = 8

NSUB = 16
NCORE = 2
C = 128
NPAD = 50048
ROWZ = NPAD // NSUB
ZR = 184
ZC = ROWZ // ZR
S15 = 3128
SLAST = N - 15 * S15

CH1 = 400
KIN1 = 16
BLK1 = 25
CH3 = 200
KIN3 = 8
BLK3 = 25

RB = 1000

_mesh = plsc.VectorSubcoreMesh(
    core_axis_name="c", subcore_axis_name="s", num_cores=NCORE, num_subcores=NSUB
)
_sc_params = pltpu.CompilerParams(use_tc_tiling_on_sc=False)

_f32 = jnp.float32


def _prep_edges(row, tiles, chunks, is_dst):
    per = E // tiles
    e = row.reshape(tiles, per)
    if is_dst:
        fill = (N + jnp.arange(tiles, dtype=jnp.int32) % (NPAD - N))[:, None]
    else:
        fill = jnp.zeros((tiles, 1), jnp.int32)
    pad = jnp.broadcast_to(fill, (tiles, chunks * C - per))
    return jnp.concatenate([e, pad], axis=1).reshape(tiles, chunks, C)


def _mm_body(x_ref, w_ref, ym0_ref, ym1_ref, zs_ref):
    y = jnp.dot(x_ref[...], w_ref[...], preferred_element_type=_f32)
    ym0_ref[...] = y[:, :HH]
    ym1_ref[...] = y[:, HH:H]
    zs_ref[...] = y[:, H:]


def _transform(x, Wm, Ws):
    din = x.shape[1]
    w = jnp.concatenate([Wm, Ws], axis=1)
    return pl.pallas_call(
        _mm_body,
        grid=(N // RB,),
        in_specs=[
            pl.BlockSpec((RB, din), lambda i: (i, 0)),
            pl.BlockSpec((din, 2 * H), lambda i: (0, 0)),
        ],
        out_specs=[
            pl.BlockSpec((RB, HH), lambda i: (i, 0)),
            pl.BlockSpec((RB, HH), lambda i: (i, 0)),
            pl.BlockSpec((RB, H), lambda i: (i, 0)),
        ],
        out_shape=[
            jax.ShapeDtypeStruct((N, HH), _f32),
            jax.ShapeDtypeStruct((N, HH), _f32),
            jax.ShapeDtypeStruct((N, H), _f32),
        ],
    )(x, w)


def _comb_body(a0_ref, a1_ref, cnt_ref, zs_ref, b_ref, w_ref,
               ym0_ref, ym1_ref, zs_ref_o):
    inv = 1.0 / jnp.maximum(cnt_ref[...][:, 0:1], 1.0)
    acc = jnp.concatenate([a0_ref[...], a1_ref[...]], axis=1)
    x = jnp.maximum(acc * inv + zs_ref[...] + b_ref[...], 0.0)
    y = jnp.dot(x, w_ref[...], preferred_element_type=_f32)
    ym0_ref[...] = y[:, :HH]
    ym1_ref[...] = y[:, HH:H]
    zs_ref_o[...] = y[:, H:]


def _combine(a0, a1, cnt, zs, b, Wm, Ws):
    w = jnp.concatenate([Wm, Ws], axis=1)
    return pl.pallas_call(
        _comb_body,
        grid=(N // RB,),
        in_specs=[
            pl.BlockSpec((RB, HH), lambda i: (i, 0)),
            pl.BlockSpec((RB, HH), lambda i: (i, 0)),
            pl.BlockSpec((RB, HH), lambda i: (i, 0)),
            pl.BlockSpec((RB, H), lambda i: (i, 0)),
            pl.BlockSpec((1, H), lambda i: (0, 0)),
            pl.BlockSpec((H, 2 * H), lambda i: (0, 0)),
        ],
        out_specs=[
            pl.BlockSpec((RB, HH), lambda i: (i, 0)),
            pl.BlockSpec((RB, HH), lambda i: (i, 0)),
            pl.BlockSpec((RB, H), lambda i: (i, 0)),
        ],
        out_shape=[
            jax.ShapeDtypeStruct((N, HH), _f32),
            jax.ShapeDtypeStruct((N, HH), _f32),
            jax.ShapeDtypeStruct((N, H), _f32),
        ],
    )(a0, a1, cnt, zs, b.reshape(1, H), w)


def _final_body(p00_ref, p01_ref, p10_ref, p11_ref, cnt_ref, zs_ref, b_ref,
                w_ref, bf_ref, out_ref):
    inv = 1.0 / jnp.maximum(cnt_ref[...][:, 0:1], 1.0)
    acc = jnp.concatenate(
        [p00_ref[...] + p10_ref[...], p01_ref[...] + p11_ref[...]], axis=1
    )
    x = jnp.maximum(acc * inv + zs_ref[...] + b_ref[...], 0.0)
    out_ref[...] = jnp.dot(x, w_ref[...], preferred_element_type=_f32) + bf_ref[...]


def _final(p00, p01, p10, p11, cnt, zs, b, Wfin, bfin):
    wp = jnp.pad(Wfin, ((0, 0), (0, OUTP - OUT)))
    bp = jnp.pad(bfin, (0, OUTP - OUT)).reshape(1, OUTP)
    out = pl.pallas_call(
        _final_body,
        grid=(N // RB,),
        in_specs=[
            pl.BlockSpec((RB, HH), lambda i: (i, 0)),
            pl.BlockSpec((RB, HH), lambda i: (i, 0)),
            pl.BlockSpec((RB, HH), lambda i: (i, 0)),
            pl.BlockSpec((RB, HH), lambda i: (i, 0)),
            pl.BlockSpec((RB, HH), lambda i: (i, 0)),
            pl.BlockSpec((RB, H), lambda i: (i, 0)),
            pl.BlockSpec((1, H), lambda i: (0, 0)),
            pl.BlockSpec((H, OUTP), lambda i: (0, 0)),
            pl.BlockSpec((1, OUTP), lambda i: (0, 0)),
        ],
        out_specs=pl.BlockSpec((RB, OUTP), lambda i: (i, 0)),
        out_shape=jax.ShapeDtypeStruct((N, OUTP), _f32),
    )(p00, p01, p10, p11, cnt, zs, b.reshape(1, H), wp, bp)
    return out[:, :OUT]


def _zero_zbuf(zbuf, rows):
    z16 = jnp.zeros((16,), _f32)

    def body(i, _):
        zbuf[i] = z16
        return 0

    lax.fori_loop(0, rows, body, 0)


def _zero_acc(zbuf, accS, base):
    def body(k, _):
        off = pl.multiple_of(base + k * ZR, 8)
        pltpu.sync_copy(zbuf, accS.at[pl.ds(off, ZR)])
        return 0

    lax.fori_loop(0, ZC, body, 0)


def _copy_out(accS, out, base):
    def body(k, _):
        off = pl.multiple_of(base + k * ZR, 8)
        pltpu.sync_copy(accS.at[pl.ds(off, ZR)], out.at[pl.ds(off, ZR)])
        return 0

    lax.fori_loop(0, ZC, body, 0)


def _edge_pass(tab, srcR, dstR, accS, srcb, dstb, rows0, rows1, sem0, sem1,
               tile, nblk, kin):
    rows = (rows0, rows1)
    sems = (sem0, sem1)

    def blk(b, _):
        pltpu.sync_copy(srcR.at[tile, pl.ds(b * kin, kin)], srcb)
        pltpu.sync_copy(dstR.at[tile, pl.ds(b * kin, kin)], dstb)
        descs = [None, None]
        descs[0] = pltpu.async_copy(tab.at[srcb.at[0]], rows[0], sems[0])
        for j in range(kin):
            nj = j + 1
            if nj < kin:
                descs[nj % 2] = pltpu.async_copy(
                    tab.at[srcb.at[nj]], rows[nj % 2], sems[nj % 2]
                )
            descs[j % 2].wait()
            pltpu.sync_copy(rows[j % 2], accS.at[dstb.at[j]], add=True)
        return 0

    lax.fori_loop(0, nblk, blk, 0)


def _half_pass(ym_h, srcR, dstR, out_h, accS, srcb, dstb, rows0, rows1,
               zbuf, sem0, sem1, s, tile, nblk, kin):
    base = s * ROWZ
    _zero_acc(zbuf, accS, base)
    plsc.subcore_barrier()
    _edge_pass(ym_h, srcR, dstR, accS, srcb, dstb, rows0, rows1, sem0, sem1,
               tile, nblk, kin)
    plsc.subcore_barrier()
    _copy_out(accS, out_h, base)


@functools.partial(
    pl.kernel,
    out_type=(
        jax.ShapeDtypeStruct((NPAD, HH), _f32),
        jax.ShapeDtypeStruct((NPAD, HH), _f32),
    ),
    mesh=_mesh,
    compiler_params=_sc_params,
    scratch_types=[
        pltpu.VMEM((KIN1, C), jnp.int32),
        pltpu.VMEM((C, HH), _f32),
        pltpu.VMEM((ZR, HH), _f32),
        pltpu.VMEM_SHARED((NPAD, HH), _f32),
    ],
)
def _counts_kernel(dst_in, dst_out, cnt_t, cnt_a, dstb, ones_v, zbuf, cntS):
    c = lax.axis_index("c")
    s = lax.axis_index("s")
    base = s * ROWZ

    _zero_zbuf(zbuf, ZR)
    o16 = jnp.ones((16,), _f32)

    def fill_ones(i, _):
        ones_v[i] = o16
        return 0

    lax.fori_loop(0, C, fill_ones, 0)
    _zero_acc(zbuf, cntS, base)
    plsc.subcore_barrier()

    def count_rel(dstR, out):
        def blk(b, _):
            pltpu.sync_copy(dstR.at[s, pl.ds(b * KIN1, KIN1)], dstb)
            for j in range(KIN1):
                pltpu.sync_copy(ones_v, cntS.at[dstb.at[j]], add=True)
            return 0

        lax.fori_loop(0, BLK1, blk, 0)
        plsc.subcore_barrier()
        _copy_out(cntS, out, base)

    @pl.when(c == 0)
    def _():
        count_rel(dst_in, cnt_t)

    @pl.when(c == 1)
    def _():
        count_rel(dst_out, cnt_a)


@functools.partial(
    pl.kernel,
    out_type=tuple(
        jax.ShapeDtypeStruct((NPAD, HH), _f32) for _ in range(4)
    ),
    mesh=_mesh,
    compiler_params=_sc_params,
    scratch_types=[
        pltpu.VMEM((KIN1, C), jnp.int32),
        pltpu.VMEM((KIN1, C), jnp.int32),
        pltpu.VMEM((C, HH), _f32),
        pltpu.VMEM((C, HH), _f32),
        pltpu.VMEM((ZR, HH), _f32),
        pltpu.VMEM_SHARED((NPAD, HH), _f32),
        pltpu.SemaphoreType.DMA,
        pltpu.SemaphoreType.DMA,
    ],
)
def _segsum2(ym_a0, ym_a1, src_in, dst_in, ym_t0, ym_t1, src_out, dst_out,
             acc_t0, acc_t1, acc_a0, acc_a1,
             srcb, dstb, rows0, rows1, zbuf, accS, sem0, sem1):
    c = lax.axis_index("c")
    s = lax.axis_index("s")

    _zero_zbuf(zbuf, ZR)

    def rel(ym0, ym1, srcR, dstR, out0, out1):
        for ym_h, out_h in ((ym0, out0), (ym1, out1)):
            _half_pass(ym_h, srcR, dstR, out_h, accS, srcb, dstb,
                       rows0, rows1, zbuf, sem0, sem1, s, s, BLK1, KIN1)

    @pl.when(c == 0)
    def _():
        rel(ym_a0, ym_a1, src_in, dst_in, acc_t0, acc_t1)

    @pl.when(c == 1)
    def _():
        rel(ym_t0, ym_t1, src_out, dst_out, acc_a0, acc_a1)


@functools.partial(
    pl.kernel,
    out_type=tuple(
        jax.ShapeDtypeStruct((NPAD, HH), _f32) for _ in range(4)
    ),
    mesh=_mesh,
    compiler_params=_sc_params,
    scratch_types=[
        pltpu.VMEM((KIN3, C), jnp.int32),
        pltpu.VMEM((KIN3, C), jnp.int32),
        pltpu.VMEM((C, HH), _f32),
        pltpu.VMEM((C, HH), _f32),
        pltpu.VMEM((ZR, HH), _f32),
        pltpu.VMEM_SHARED((NPAD, HH), _f32),
        pltpu.SemaphoreType.DMA,
        pltpu.SemaphoreType.DMA,
    ],
)
def _segsum1(ym0, ym1, src3, dst3, p00, p01, p10, p11,
             srcb, dstb, rows0, rows1, zbuf, accS, sem0, sem1):
    c = lax.axis_index("c")
    s = lax.axis_index("s")
    tile = c * NSUB + s

    _zero_zbuf(zbuf, ZR)

    def run(out0, out1):
        for ym_h, out_h in ((ym0, out0), (ym1, out1)):
            _half_pass(ym_h, src3, dst3, out_h, accS, srcb, dstb,
                       rows0, rows1, zbuf, sem0, sem1, s, tile, BLK3, KIN3)

    @pl.when(c == 0)
    def _():
        run(p00, p01)

    @pl.when(c == 1)
    def _():
        run(p10, p11)


def kernel(x_addr, x_tx, edge_index_input, edge_index_output,
           Wl1i, Wr1i, bl1i, Wl1o, Wr1o, bl1o,
           Wl2i, Wr2i, bl2i, Wl2o, Wr2o, bl2o,
           Wl3i, Wr3i, bl3i, Wl3o, Wr3o, bl3o,
           Wfin, bfin):
    ei = edge_index_input.astype(jnp.int32)
    eo = edge_index_output.astype(jnp.int32)
    src_in = _prep_edges(ei[0], NSUB, CH1, False)
    dst_in = _prep_edges(ei[1], NSUB, CH1, True)
    src_out = _prep_edges(eo[0], NSUB, CH1, False)
    dst_out = _prep_edges(eo[1], NSUB, CH1, True)
    src3 = _prep_edges(eo[0], NCORE * NSUB, CH3, False)
    dst3 = _prep_edges(eo[1], NCORE * NSUB, CH3, True)

    cnt_t, cnt_a = _counts_kernel(dst_in, dst_out)

    ym_a0, ym_a1, zs_a1 = _transform(x_addr, Wl1i, Wr1o)
    ym_t0, ym_t1, zs_t1 = _transform(x_tx, Wl1o, Wr1i)
    acc_t0, acc_t1, acc_a0, acc_a1 = _segsum2(
        ym_a0, ym_a1, src_in, dst_in, ym_t0, ym_t1, src_out, dst_out
    )

    ym_t0, ym_t1, zs_t2 = _combine(acc_t0, acc_t1, cnt_t, zs_t1, bl1i, Wl2o, Wr2i)
    ym_a0, ym_a1, zs_a2 = _combine(acc_a0, acc_a1, cnt_a, zs_a1, bl1o, Wl2i, Wr2o)
    acc_t0, acc_t1, acc_a0, acc_a1 = _segsum2(
        ym_a0, ym_a1, src_in, dst_in, ym_t0, ym_t1, src_out, dst_out
    )

    ym_t0, ym_t1, _ = _combine(acc_t0, acc_t1, cnt_t, zs_t2, bl2i, Wl3o, Wr3i)
    _, _, zs_a3 = _combine(acc_a0, acc_a1, cnt_a, zs_a2, bl2o, Wl3i, Wr3o)
    p00, p01, p10, p11 = _segsum1(ym_t0, ym_t1, src3, dst3)

    return _final(p00, p01, p10, p11, cnt_a, zs_a3, bl3o, Wfin, bfin)
